# baseline (speedup 1.0000x reference)
import functools

import jax
import jax.numpy as jnp
from jax import lax
from jax.experimental import pallas as pl
from jax.experimental.pallas import tpu as pltpu
from jax.experimental.pallas import tpu_sc as plsc

N = 3200000
M = 100000
MP = 100352
NB = N // 128
NW = 32
CH = 64
SL = MP // 16

_mesh = plsc.VectorSubcoreMesh(core_axis_name="c", subcore_axis_name="s")
_params = pltpu.CompilerParams(use_tc_tiling_on_sc=False, needs_layout_passes=False)
_f32 = jnp.float32
_i32 = jnp.int32


def _worker_range(w):
    lo = (w * NB) // NW
    hi = ((w + 1) * NB) // NW
    return lo, hi


@functools.partial(
    pl.kernel,
    out_type=jax.ShapeDtypeStruct((2, MP, 8), _f32),
    mesh=_mesh,
    compiler_params=_params,
    scratch_types=[
        pltpu.VMEM_SHARED((MP, 8), _f32),
        pltpu.VMEM((CH, 1, 128), _i32),
        pltpu.VMEM((CH * 128,), _f32),
        pltpu.VMEM((CH * 128,), _f32),
        pltpu.VMEM((CH * 128,), _f32),
        pltpu.VMEM((128, 8), _f32),
        pltpu.VMEM((128, 8), _f32),
        pltpu.VMEM((128, 8), _f32),
        pltpu.VMEM((128, 8), _f32),
        pltpu.SemaphoreType.DMA,
        pltpu.SemaphoreType.DMA,
        pltpu.SemaphoreType.DMA,
        pltpu.SemaphoreType.DMA,
    ],
)
def _k_partials(xs, ys, zs, ids3, zeros8,
                pout, acc, ids_v, xs_v, ys_v, zs_v,
                vals_a, vals_b, vals_c, vals_d, sa, sb, sc_, sd):
    c = lax.axis_index("c")
    s = lax.axis_index("s")
    w = c * 16 + s

    zsl = pl.ds(s * SL, SL)
    pltpu.sync_copy(zeros8.at[zsl], acc.at[zsl])
    plsc.subcore_barrier()

    lane = lax.iota(_i32, 16)
    c0 = jnp.zeros((16,), _i32)
    c1 = jnp.ones((16,), _i32)
    c2 = jnp.full((16,), 2, _i32)
    c3 = jnp.full((16,), 3, _i32)
    onesv = jnp.ones((16,), _f32)

    lo, hi = _worker_range(w)
    nf = (hi - lo) // CH

    for vals_v in (vals_a, vals_b, vals_c, vals_d):
        for t in range(8):
            plsc.store_scatter(vals_v, [lane + 16 * t, c3], onesv)

    def build(j, vals_v):
        abase = j * 128
        for t in range(8):
            iv = lane + 16 * t
            sl = pl.ds(abase + 16 * t, 16)
            plsc.store_scatter(vals_v, [iv, c0], xs_v[sl])
            plsc.store_scatter(vals_v, [iv, c1], ys_v[sl])
            plsc.store_scatter(vals_v, [iv, c2], zs_v[sl])

    def chunk(k, _):
        base = lo + k * CH
        asl = pl.ds(base * 128, CH * 128)
        pltpu.sync_copy(ids3.at[pl.ds(base, CH)], ids_v)
        pltpu.sync_copy(xs.at[asl], xs_v)
        pltpu.sync_copy(ys.at[asl], ys_v)
        pltpu.sync_copy(zs.at[asl], zs_v)

        def blk(p, _):
            for jo, vals_v, sem in ((0, vals_a, sa), (1, vals_b, sb),
                                    (2, vals_c, sc_), (3, vals_d, sd)):
                j = 4 * p + jo

                @pl.when(p > 0)
                def _():
                    pltpu.make_async_copy(
                        vals_v, acc.at[ids_v.at[j, 0]], sem).wait()

                build(j, vals_v)
                pltpu.async_copy(vals_v, acc.at[ids_v.at[j, 0]], sem, add=True)
            return 0

        lax.fori_loop(0, CH // 4, blk, 0)
        pltpu.make_async_copy(vals_a, acc.at[ids_v.at[CH - 4, 0]], sa).wait()
        pltpu.make_async_copy(vals_b, acc.at[ids_v.at[CH - 3, 0]], sb).wait()
        pltpu.make_async_copy(vals_c, acc.at[ids_v.at[CH - 2, 0]], sc_).wait()
        pltpu.make_async_copy(vals_d, acc.at[ids_v.at[CH - 1, 0]], sd).wait()
        return 0

    lax.fori_loop(0, nf, chunk, 0)

    tl = lo + nf * CH

    def tblk(j, _):
        bsl = pl.ds((tl + j) * 128, 128)
        sl0 = pl.ds(0, 128)
        pltpu.sync_copy(ids3.at[pl.ds(tl + j, 1)], ids_v.at[pl.ds(0, 1)])
        pltpu.sync_copy(xs.at[bsl], xs_v.at[sl0])
        pltpu.sync_copy(ys.at[bsl], ys_v.at[sl0])
        pltpu.sync_copy(zs.at[bsl], zs_v.at[sl0])
        build(0, vals_a)
        pltpu.sync_copy(vals_a, acc.at[ids_v.at[0, 0]], add=True)
        return 0

    lax.fori_loop(0, hi - tl, tblk, 0)

    plsc.subcore_barrier()
    pltpu.sync_copy(acc.at[zsl], pout.at[c, zsl])


W = 2048


@functools.partial(
    pl.kernel,
    out_type=(
        jax.ShapeDtypeStruct((N,), _f32),
        jax.ShapeDtypeStruct((N,), _f32),
        jax.ShapeDtypeStruct((N,), _f32),
    ),
    mesh=_mesh,
    compiler_params=_params,
    scratch_types=[
        pltpu.VMEM((CH, 1, 128), _i32),
        pltpu.VMEM((CH * 128,), _f32),
        pltpu.VMEM((CH * 128,), _f32),
        pltpu.VMEM((CH * 128,), _f32),
        pltpu.VMEM((CH * 128,), _f32),
        pltpu.VMEM((CH * 128,), _f32),
        pltpu.VMEM((CH * 128,), _f32),
        pltpu.VMEM((W, 8), _f32),
        pltpu.VMEM((W, 8), _f32),
        pltpu.VMEM((128, 8), _f32),
        pltpu.VMEM((128, 8), _f32),
        pltpu.SemaphoreType.DMA,
        pltpu.SemaphoreType.DMA,
    ],
)
def _k_center(xs, ys, zs, ids3, part8, ox, oy, oz,
              ids_v, xs_v, ys_v, zs_v, ox_v, oy_v, oz_v,
              pa, pb, mra, mrb, sa, sb):
    c = lax.axis_index("c")
    s = lax.axis_index("s")
    w = c * 16 + s

    lane = lax.iota(_i32, 16)
    c0 = jnp.zeros((16,), _i32)
    c1 = jnp.ones((16,), _i32)
    c2 = jnp.full((16,), 2, _i32)
    c3 = jnp.full((16,), 3, _i32)
    pr2 = lane // 8
    pcol = lane - 8 * pr2

    lo, hi = _worker_range(w)
    nc = (hi - lo + CH - 1) // CH

    def chunk(k, _):
        base = jnp.minimum(lo + k * CH, hi - CH)
        asl = pl.ds(base * 128, CH * 128)
        pltpu.sync_copy(ids3.at[pl.ds(base, CH)], ids_v)
        pltpu.sync_copy(xs.at[asl], xs_v)
        pltpu.sync_copy(ys.at[asl], ys_v)
        pltpu.sync_copy(zs.at[asl], zs_v)

        start = jnp.minimum(ids_v[0, 0, pl.ds(0, 16)][0], MP - W)
        need = ids_v[CH - 1, 0, pl.ds(112, 16)][15] - start + 1

        @pl.when(need <= W)
        def _fast():
            psl = pl.ds(start, W)
            pltpu.sync_copy(part8.at[0, psl, :], pa)
            pltpu.sync_copy(part8.at[1, psl, :], pb)

            def comb(t, _):
                rows = pr2 + 2 * t
                v = (plsc.load_gather(pa, [rows, pcol])
                     + plsc.load_gather(pb, [rows, pcol]))
                plsc.store_scatter(pa, [rows, pcol], v)
                cnt = plsc.load_gather(pa, [rows, c3])
                plsc.store_scatter(pa, [rows, pcol], v / jnp.maximum(cnt, 1.0))
                return 0

            lax.fori_loop(0, (need + 1) // 2, comb, 0)

            def blk(j, _):
                abase = j * 128
                for t in range(8):
                    sl = pl.ds(abase + t * 16, 16)
                    rel = ids_v[j, 0, pl.ds(16 * t, 16)] - start
                    ox_v[sl] = xs_v[sl] - plsc.load_gather(pa, [rel, c0])
                    oy_v[sl] = ys_v[sl] - plsc.load_gather(pa, [rel, c1])
                    oz_v[sl] = zs_v[sl] - plsc.load_gather(pa, [rel, c2])
                return 0

            lax.fori_loop(0, CH, blk, 0)

        @pl.when(need > W)
        def _slow():
            def blk(j, _):
                da = pltpu.async_copy(part8.at[0].at[ids_v.at[j, 0]], mra, sa)
                db = pltpu.async_copy(part8.at[1].at[ids_v.at[j, 0]], mrb, sb)
                da.wait()
                db.wait()
                abase = j * 128
                for t in range(8):
                    sl = pl.ds(abase + t * 16, 16)
                    iv = lane + 16 * t
                    cnt = (plsc.load_gather(mra, [iv, c3])
                           + plsc.load_gather(mrb, [iv, c3]))
                    inv = 1.0 / jnp.maximum(cnt, 1.0)
                    mxv = (plsc.load_gather(mra, [iv, c0])
                           + plsc.load_gather(mrb, [iv, c0])) * inv
                    myv = (plsc.load_gather(mra, [iv, c1])
                           + plsc.load_gather(mrb, [iv, c1])) * inv
                    mzv = (plsc.load_gather(mra, [iv, c2])
                           + plsc.load_gather(mrb, [iv, c2])) * inv
                    ox_v[sl] = xs_v[sl] - mxv
                    oy_v[sl] = ys_v[sl] - myv
                    oz_v[sl] = zs_v[sl] - mzv
                return 0

            lax.fori_loop(0, CH, blk, 0)

        pltpu.sync_copy(ox_v, ox.at[asl])
        pltpu.sync_copy(oy_v, oy.at[asl])
        pltpu.sync_copy(oz_v, oz.at[asl])
        return 0

    lax.fori_loop(0, nc, chunk, 0)


def kernel(atoms_x, graph_batch):
    xs = atoms_x[:, 0]
    ys = atoms_x[:, 1]
    zs = atoms_x[:, 2]
    ids3 = graph_batch.reshape(NB, 1, 128)
    zeros8 = jnp.zeros((MP, 8), _f32)

    partial = _k_partials(xs, ys, zs, ids3, zeros8)
    ox, oy, oz = _k_center(xs, ys, zs, ids3, partial)
    return jnp.stack([ox, oy, oz], axis=1)
